# baseline (speedup 1.0000x reference)
import functools

import jax
import jax.numpy as jnp
from jax import lax
from jax.experimental import pallas as pl
from jax.experimental.pallas import tpu as pltpu
from jax.experimental.pallas import tpu_sc as plsc

F32 = jnp.float32
I32 = jnp.int32

NC = 2
NS = 16
NW = NC * NS
L = 16

CH = 128


def _sc_mesh():
    return plsc.VectorSubcoreMesh(core_axis_name="c", subcore_axis_name="s")


NT = L


@functools.lru_cache(maxsize=None)
def _make_deg_kernel(N, EP):
    assert EP % (CH * NW) == 0 and N % NS == 0
    CPT = EP // CH // NW
    RPT = N // NS
    nfull = RPT // CH
    rem = RPT % CH

    @functools.partial(
        pl.kernel,
        mesh=_sc_mesh(),
        out_type=jax.ShapeDtypeStruct((NC * N, L), F32),
        compiler_params=pltpu.CompilerParams(use_tc_tiling_on_sc=False),
        scratch_types=[
            pltpu.VMEM((CPT, CH), I32),
            pltpu.VMEM((CH, L), F32),
            pltpu.VMEM_SHARED((N + NT, L), F32),
            pltpu.SemaphoreType.DMA,
        ],
    )
    def deg_kernel(row_hbm, out_hbm, ridx, ones_v, acc, sem):
        c = lax.axis_index("c")
        s = lax.axis_index("s")
        t = c * NS + s
        zeros = jnp.zeros((L,), F32)
        ones = jnp.ones((L,), F32)

        pltpu.sync_copy(row_hbm.at[pl.ds(t * CPT, CPT)], ridx)

        def zf(i, carry):
            ones_v[i, pl.ds(0, L)] = zeros
            return carry

        lax.fori_loop(0, CH, zf, None)
        for q in range(nfull):
            pltpu.sync_copy(ones_v, acc.at[pl.ds(s * RPT + q * CH, CH)])
        if rem:
            pltpu.sync_copy(ones_v.at[pl.ds(0, rem)],
                            acc.at[pl.ds(s * RPT + nfull * CH, rem)])
        if NT:
            @pl.when(s == 0)
            def _():
                pltpu.sync_copy(ones_v.at[pl.ds(0, NT)],
                                acc.at[pl.ds(N, NT)])

        def of(i, carry):
            ones_v[i, pl.ds(0, L)] = ones
            return carry

        lax.fori_loop(0, CH, of, None)
        plsc.subcore_barrier()

        def fire(k, carry):
            pltpu.async_copy(ones_v, acc.at[ridx.at[k]], sem, add=True)
            return carry

        lax.fori_loop(0, CPT, fire, None)

        def drain(k, carry):
            pltpu.make_async_copy(ones_v, acc.at[ridx.at[0]], sem).wait()
            return carry

        lax.fori_loop(0, CPT, drain, None)
        plsc.subcore_barrier()

        for q in range(nfull):
            pltpu.sync_copy(acc.at[pl.ds(s * RPT + q * CH, CH)], ones_v)
            pltpu.sync_copy(ones_v,
                            out_hbm.at[pl.ds(c * N + s * RPT + q * CH, CH)])
        if rem:
            pltpu.sync_copy(acc.at[pl.ds(s * RPT + nfull * CH, rem)],
                            ones_v.at[pl.ds(0, rem)])
            pltpu.sync_copy(
                ones_v.at[pl.ds(0, rem)],
                out_hbm.at[pl.ds(c * N + s * RPT + nfull * CH, rem)])

    return deg_kernel


@functools.lru_cache(maxsize=None)
def _make_spmm_kernel(N, EP, D):
    GRP = 2 if D >= 128 else 4
    NH = 2
    assert EP % (CH * NW) == 0 and D % L == 0 and N % NS == 0
    CPT = EP // CH // NW
    HC = CPT // NH
    assert CPT % NH == 0 and HC % GRP == 0
    RPT = N // NS
    nfull = RPT // CH
    rem = RPT % CH

    @functools.partial(
        pl.kernel,
        mesh=_sc_mesh(),
        out_type=jax.ShapeDtypeStruct((NC * N, D), F32),
        compiler_params=pltpu.CompilerParams(use_tc_tiling_on_sc=False),
        scratch_types=[
            pltpu.VMEM((HC, CH), I32),
            pltpu.VMEM((HC, CH), I32),
            pltpu.VMEM((HC, CH), F32),
            pltpu.VMEM((GRP, CH, D), F32),
            pltpu.VMEM_SHARED((N, D), F32),
        ] + [pltpu.SemaphoreType.DMA] * GRP,
    )
    def spmm_kernel(z_hbm, row_hbm, col_hbm, ew_hbm, out_hbm,
                    ridx, cidx, ewv, rows, acc, *sems):
        c = lax.axis_index("c")
        s = lax.axis_index("s")
        t = c * NS + s
        zeros = jnp.zeros((L,), F32)

        def zr(i, carry):
            for j in range(D // L):
                rows[0, i, pl.ds(j * L, L)] = zeros
            return carry

        lax.fori_loop(0, CH, zr, None)

        for q in range(nfull):
            pltpu.sync_copy(rows.at[0], acc.at[pl.ds(s * RPT + q * CH, CH)])
        if rem:
            pltpu.sync_copy(rows.at[0].at[pl.ds(0, rem)],
                            acc.at[pl.ds(s * RPT + nfull * CH, rem)])
        plsc.subcore_barrier()

        def scale_rows(b, k):
            def scale(g, c2):
                wv = ewv[k, pl.ds(g * L, L)]
                for lidx in range(L):
                    w = wv[lidx]
                    r = g * L + lidx
                    for j in range(D // L):
                        rows[b, r, pl.ds(j * L, L)] = \
                            rows[b, r, pl.ds(j * L, L)] * w
                return c2

            lax.fori_loop(0, CH // L, scale, None)

        def group_body(gq, carry):
            k0 = gq * GRP
            ds = [pltpu.async_copy(z_hbm.at[ridx.at[k0 + b]], rows.at[b],
                                   sems[b]) for b in range(GRP)]
            for b in range(GRP):
                ds[b].wait()
                scale_rows(b, k0 + b)
                pltpu.sync_copy(rows.at[b], acc.at[cidx.at[k0 + b]],
                                add=True)
            return carry

        for h in range(NH):
            start = t * CPT + h * HC
            pltpu.sync_copy(row_hbm.at[pl.ds(start, HC)], ridx)
            pltpu.sync_copy(col_hbm.at[pl.ds(start, HC)], cidx)
            pltpu.sync_copy(ew_hbm.at[pl.ds(start, HC)], ewv)
            lax.fori_loop(0, HC // GRP, group_body, None)
        plsc.subcore_barrier()

        for q in range(nfull):
            pltpu.sync_copy(acc.at[pl.ds(s * RPT + q * CH, CH)], rows.at[0])
            pltpu.sync_copy(rows.at[0],
                            out_hbm.at[pl.ds(c * N + s * RPT + q * CH, CH)])
        if rem:
            pltpu.sync_copy(acc.at[pl.ds(s * RPT + nfull * CH, rem)],
                            rows.at[0].at[pl.ds(0, rem)])
            pltpu.sync_copy(
                rows.at[0].at[pl.ds(0, rem)],
                out_hbm.at[pl.ds(c * N + s * RPT + nfull * CH, rem)])

    return spmm_kernel


_ROWS = 1000


def _tc_matmul(x, W):
    N, Din = x.shape
    K = W.shape[1]

    def body(x_ref, w_ref, o_ref):
        o_ref[...] = jnp.dot(x_ref[...], w_ref[...],
                             preferred_element_type=F32)

    return pl.pallas_call(
        body,
        grid=(N // _ROWS,),
        in_specs=[pl.BlockSpec((_ROWS, Din), lambda i: (i, 0)),
                  pl.BlockSpec((Din, K), lambda i: (0, 0))],
        out_specs=pl.BlockSpec((_ROWS, K), lambda i: (i, 0)),
        out_shape=jax.ShapeDtypeStruct((N, K), F32),
    )(x, W)


def _tc_scale(degp0, degp1, y1):
    N, D = y1.shape

    def body(d0_ref, d1_ref, y_ref, z_ref, dis_ref):
        d = d0_ref[...] + d1_ref[...] + 1.0
        dis = lax.rsqrt(d)[:, 0:1]
        dis_ref[...] = dis
        z_ref[...] = y_ref[...] * dis

    return pl.pallas_call(
        body,
        grid=(N // _ROWS,),
        in_specs=[pl.BlockSpec((_ROWS, L), lambda i: (i, 0)),
                  pl.BlockSpec((_ROWS, L), lambda i: (i, 0)),
                  pl.BlockSpec((_ROWS, D), lambda i: (i, 0))],
        out_specs=[pl.BlockSpec((_ROWS, D), lambda i: (i, 0)),
                   pl.BlockSpec((_ROWS, 1), lambda i: (i, 0))],
        out_shape=[jax.ShapeDtypeStruct((N, D), F32),
                   jax.ShapeDtypeStruct((N, 1), F32)],
    )(degp0, degp1, y1)


def _tc_layer2(dis, pa, pb, z1, b1, W2):
    N, D = z1.shape
    K = W2.shape[1]

    def body(dis_ref, pa_ref, pb_ref, z1_ref, b1_ref, w_ref, z2_ref):
        dis = dis_ref[...]
        h = (pa_ref[...] + pb_ref[...] + z1_ref[...]) * dis + b1_ref[...]
        h = jnp.maximum(h, 0.0)
        z2_ref[...] = jnp.dot(h, w_ref[...], preferred_element_type=F32) * dis

    return pl.pallas_call(
        body,
        grid=(N // _ROWS,),
        in_specs=[pl.BlockSpec((_ROWS, 1), lambda i: (i, 0)),
                  pl.BlockSpec((_ROWS, D), lambda i: (i, 0)),
                  pl.BlockSpec((_ROWS, D), lambda i: (i, 0)),
                  pl.BlockSpec((_ROWS, D), lambda i: (i, 0)),
                  pl.BlockSpec((1, D), lambda i: (0, 0)),
                  pl.BlockSpec((D, K), lambda i: (0, 0))],
        out_specs=pl.BlockSpec((_ROWS, K), lambda i: (i, 0)),
        out_shape=jax.ShapeDtypeStruct((N, K), F32),
    )(dis, pa, pb, z1, b1, W2)


def _tc_out(dis, pa, pb, z2, b2):
    N, K = z2.shape

    def body(dis_ref, pa_ref, pb_ref, z2_ref, b2_ref, o_ref):
        v = (pa_ref[...] + pb_ref[...] + z2_ref[...]) * dis_ref[...] \
            + b2_ref[...]
        v = v - jnp.max(v, axis=1, keepdims=True)
        o_ref[...] = v - jnp.log(jnp.sum(jnp.exp(v), axis=1, keepdims=True))

    return pl.pallas_call(
        body,
        grid=(N // _ROWS,),
        in_specs=[pl.BlockSpec((_ROWS, 1), lambda i: (i, 0)),
                  pl.BlockSpec((_ROWS, K), lambda i: (i, 0)),
                  pl.BlockSpec((_ROWS, K), lambda i: (i, 0)),
                  pl.BlockSpec((_ROWS, K), lambda i: (i, 0)),
                  pl.BlockSpec((1, K), lambda i: (0, 0))],
        out_specs=pl.BlockSpec((_ROWS, K), lambda i: (i, 0)),
        out_shape=jax.ShapeDtypeStruct((N, K), F32),
    )(dis, pa, pb, z2, b2)


def kernel(x, edge_index, edge_attr, W1, b1, W2, b2):
    N, Din = x.shape
    E = edge_index.shape[1]
    Dh = W1.shape[1]
    Dout = W2.shape[1]

    row = edge_index[0]
    col = edge_index[1]
    EP = -(-E // (CH * NW * 8)) * (CH * NW * 8)
    pad = EP - E
    if pad:
        zpad_i = jnp.zeros((pad,), I32)
        row_sp = jnp.concatenate([row, zpad_i]).reshape(EP // CH, CH)
        col2 = jnp.concatenate([col, zpad_i]).reshape(EP // CH, CH)
        ew2 = jnp.concatenate(
            [edge_attr, jnp.zeros((pad,), F32)]).reshape(EP // CH, CH)
        row_dg = jnp.concatenate(
            [row, N + (jnp.arange(pad, dtype=I32) % NT)]
        ).reshape(EP // CH, CH)
    else:
        row_sp = row_dg = row.reshape(EP // CH, CH)
        col2 = col.reshape(EP // CH, CH)
        ew2 = edge_attr.reshape(EP // CH, CH)

    degp = _make_deg_kernel(N, EP)(row_dg).reshape(NC, N, L)
    y1 = _tc_matmul(x, W1)
    z1, dis = _tc_scale(degp[0], degp[1], y1)

    p1 = _make_spmm_kernel(N, EP, Dh)(
        z1, row_sp, col2, ew2).reshape(NC, N, Dh)
    z2 = _tc_layer2(dis, p1[0], p1[1], z1, b1.reshape(1, Dh), W2)

    p2 = _make_spmm_kernel(N, EP, Dout)(
        z2, row_sp, col2, ew2).reshape(NC, N, Dout)
    return _tc_out(dis, p2[0], p2[1], z2, b2.reshape(1, Dout))


# speedup vs baseline: 1.9801x; 1.9801x over previous
import functools

import jax
import jax.numpy as jnp
from jax import lax
from jax.experimental import pallas as pl
from jax.experimental.pallas import tpu as pltpu
from jax.experimental.pallas import tpu_sc as plsc

F32 = jnp.float32
I32 = jnp.int32

NC = 2
NS = 16
NW = NC * NS
L = 16

CH = 128


def _sc_mesh():
    return plsc.VectorSubcoreMesh(core_axis_name="c", subcore_axis_name="s")


NT = L


@functools.lru_cache(maxsize=None)
def _make_deg_kernel(N, EP):
    assert EP % (CH * NW) == 0 and N % NS == 0
    CPT = EP // CH // NW
    RPT = N // NS
    nfull = RPT // CH
    rem = RPT % CH

    @functools.partial(
        pl.kernel,
        mesh=_sc_mesh(),
        out_type=jax.ShapeDtypeStruct((NC * N, L), F32),
        compiler_params=pltpu.CompilerParams(use_tc_tiling_on_sc=False),
        scratch_types=[
            pltpu.VMEM((CPT, CH), I32),
            pltpu.VMEM((CH, L), F32),
            pltpu.VMEM_SHARED((N + NT, L), F32),
            pltpu.SemaphoreType.DMA,
        ],
    )
    def deg_kernel(row_hbm, out_hbm, ridx, ones_v, acc, sem):
        c = lax.axis_index("c")
        s = lax.axis_index("s")
        t = c * NS + s
        zeros = jnp.zeros((L,), F32)
        ones = jnp.ones((L,), F32)

        pltpu.sync_copy(row_hbm.at[pl.ds(t * CPT, CPT)], ridx)

        def zf(i, carry):
            ones_v[i, pl.ds(0, L)] = zeros
            return carry

        lax.fori_loop(0, CH, zf, None)
        for q in range(nfull):
            pltpu.sync_copy(ones_v, acc.at[pl.ds(s * RPT + q * CH, CH)])
        if rem:
            pltpu.sync_copy(ones_v.at[pl.ds(0, rem)],
                            acc.at[pl.ds(s * RPT + nfull * CH, rem)])
        if NT:
            @pl.when(s == 0)
            def _():
                pltpu.sync_copy(ones_v.at[pl.ds(0, NT)],
                                acc.at[pl.ds(N, NT)])

        def of(i, carry):
            ones_v[i, pl.ds(0, L)] = ones
            return carry

        lax.fori_loop(0, CH, of, None)
        plsc.subcore_barrier()

        def fire(k, carry):
            pltpu.async_copy(ones_v, acc.at[ridx.at[k]], sem, add=True)
            return carry

        lax.fori_loop(0, CPT, fire, None)

        def drain(k, carry):
            pltpu.make_async_copy(ones_v, acc.at[ridx.at[0]], sem).wait()
            return carry

        lax.fori_loop(0, CPT, drain, None)
        plsc.subcore_barrier()

        for q in range(nfull):
            pltpu.sync_copy(acc.at[pl.ds(s * RPT + q * CH, CH)], ones_v)
            pltpu.sync_copy(ones_v,
                            out_hbm.at[pl.ds(c * N + s * RPT + q * CH, CH)])
        if rem:
            pltpu.sync_copy(acc.at[pl.ds(s * RPT + nfull * CH, rem)],
                            ones_v.at[pl.ds(0, rem)])
            pltpu.sync_copy(
                ones_v.at[pl.ds(0, rem)],
                out_hbm.at[pl.ds(c * N + s * RPT + nfull * CH, rem)])

    return deg_kernel


@functools.lru_cache(maxsize=None)
def _make_spmm_kernel(N, EP, D):
    GRP = 2 if D >= 128 else 4
    NH = 2
    assert EP % (CH * NW) == 0 and D % L == 0 and N % NS == 0
    CPT = EP // CH // NW
    HC = CPT // NH
    assert CPT % NH == 0 and HC % GRP == 0
    RPT = N // NS
    nfull = RPT // CH
    rem = RPT % CH

    @functools.partial(
        pl.kernel,
        mesh=_sc_mesh(),
        out_type=jax.ShapeDtypeStruct((NC * N, D), F32),
        compiler_params=pltpu.CompilerParams(use_tc_tiling_on_sc=False),
        scratch_types=[
            pltpu.VMEM((HC, CH), I32),
            pltpu.VMEM((HC, CH), I32),
            pltpu.VMEM((HC, CH), F32),
            pltpu.VMEM((GRP, CH, D), F32),
            pltpu.VMEM_SHARED((N + NT, D), F32),
        ] + [pltpu.SemaphoreType.DMA] * GRP,
    )
    def spmm_kernel(z_hbm, row_hbm, col_hbm, ew_hbm, out_hbm,
                    ridx, cidx, ewv, rows, acc, *sems):
        c = lax.axis_index("c")
        s = lax.axis_index("s")
        t = c * NS + s
        zeros = jnp.zeros((L,), F32)

        def zr(i, carry):
            for j in range(D // L):
                rows[0, i, pl.ds(j * L, L)] = zeros
            return carry

        lax.fori_loop(0, CH, zr, None)

        for q in range(nfull):
            pltpu.sync_copy(rows.at[0], acc.at[pl.ds(s * RPT + q * CH, CH)])
        if rem:
            pltpu.sync_copy(rows.at[0].at[pl.ds(0, rem)],
                            acc.at[pl.ds(s * RPT + nfull * CH, rem)])
        plsc.subcore_barrier()

        def scale_rows(b, k):
            def scale(g, c2):
                wv = ewv[k, pl.ds(g * L, L)]
                for lidx in range(L):
                    w = wv[lidx]
                    r = g * L + lidx
                    for j in range(D // L):
                        rows[b, r, pl.ds(j * L, L)] = \
                            rows[b, r, pl.ds(j * L, L)] * w
                return c2

            lax.fori_loop(0, CH // L, scale, None)

        def group_body(gq, carry):
            k0 = gq * GRP
            ds = [pltpu.async_copy(z_hbm.at[ridx.at[k0 + b]], rows.at[b],
                                   sems[b]) for b in range(GRP)]
            for b in range(GRP):
                ds[b].wait()
                scale_rows(b, k0 + b)
                pltpu.sync_copy(rows.at[b], acc.at[cidx.at[k0 + b]],
                                add=True)
            return carry

        for h in range(NH):
            start = t * CPT + h * HC
            pltpu.sync_copy(row_hbm.at[pl.ds(start, HC)], ridx)
            pltpu.sync_copy(col_hbm.at[pl.ds(start, HC)], cidx)
            pltpu.sync_copy(ew_hbm.at[pl.ds(start, HC)], ewv)
            lax.fori_loop(0, HC // GRP, group_body, None)
        plsc.subcore_barrier()

        for q in range(nfull):
            pltpu.sync_copy(acc.at[pl.ds(s * RPT + q * CH, CH)], rows.at[0])
            pltpu.sync_copy(rows.at[0],
                            out_hbm.at[pl.ds(c * N + s * RPT + q * CH, CH)])
        if rem:
            pltpu.sync_copy(acc.at[pl.ds(s * RPT + nfull * CH, rem)],
                            rows.at[0].at[pl.ds(0, rem)])
            pltpu.sync_copy(
                rows.at[0].at[pl.ds(0, rem)],
                out_hbm.at[pl.ds(c * N + s * RPT + nfull * CH, rem)])

    return spmm_kernel


_ROWS = 1000


def _tc_matmul(x, W):
    N, Din = x.shape
    K = W.shape[1]

    def body(x_ref, w_ref, o_ref):
        o_ref[...] = jnp.dot(x_ref[...], w_ref[...],
                             preferred_element_type=F32)

    return pl.pallas_call(
        body,
        grid=(N // _ROWS,),
        in_specs=[pl.BlockSpec((_ROWS, Din), lambda i: (i, 0)),
                  pl.BlockSpec((Din, K), lambda i: (0, 0))],
        out_specs=pl.BlockSpec((_ROWS, K), lambda i: (i, 0)),
        out_shape=jax.ShapeDtypeStruct((N, K), F32),
    )(x, W)


def _tc_scale(degp0, degp1, y1):
    N, D = y1.shape

    def body(d0_ref, d1_ref, y_ref, z_ref, dis_ref):
        d = d0_ref[...] + d1_ref[...] + 1.0
        dis = lax.rsqrt(d)[:, 0:1]
        dis_ref[...] = dis
        z_ref[...] = y_ref[...] * dis

    return pl.pallas_call(
        body,
        grid=(N // _ROWS,),
        in_specs=[pl.BlockSpec((_ROWS, L), lambda i: (i, 0)),
                  pl.BlockSpec((_ROWS, L), lambda i: (i, 0)),
                  pl.BlockSpec((_ROWS, D), lambda i: (i, 0))],
        out_specs=[pl.BlockSpec((_ROWS, D), lambda i: (i, 0)),
                   pl.BlockSpec((_ROWS, 1), lambda i: (i, 0))],
        out_shape=[jax.ShapeDtypeStruct((N, D), F32),
                   jax.ShapeDtypeStruct((N, 1), F32)],
    )(degp0, degp1, y1)


def _tc_layer2(dis, pa, pb, z1, b1, W2):
    N, D = z1.shape
    K = W2.shape[1]

    def body(dis_ref, pa_ref, pb_ref, z1_ref, b1_ref, w_ref, z2_ref):
        dis = dis_ref[...]
        h = (pa_ref[...] + pb_ref[...] + z1_ref[...]) * dis + b1_ref[...]
        h = jnp.maximum(h, 0.0)
        z2_ref[...] = jnp.dot(h, w_ref[...], preferred_element_type=F32) * dis

    return pl.pallas_call(
        body,
        grid=(N // _ROWS,),
        in_specs=[pl.BlockSpec((_ROWS, 1), lambda i: (i, 0)),
                  pl.BlockSpec((_ROWS, D), lambda i: (i, 0)),
                  pl.BlockSpec((_ROWS, D), lambda i: (i, 0)),
                  pl.BlockSpec((_ROWS, D), lambda i: (i, 0)),
                  pl.BlockSpec((1, D), lambda i: (0, 0)),
                  pl.BlockSpec((D, K), lambda i: (0, 0))],
        out_specs=pl.BlockSpec((_ROWS, K), lambda i: (i, 0)),
        out_shape=jax.ShapeDtypeStruct((N, K), F32),
    )(dis, pa, pb, z1, b1, W2)


def _tc_out(dis, pa, pb, z2, b2):
    N, K = z2.shape

    def body(dis_ref, pa_ref, pb_ref, z2_ref, b2_ref, o_ref):
        v = (pa_ref[...] + pb_ref[...] + z2_ref[...]) * dis_ref[...] \
            + b2_ref[...]
        v = v - jnp.max(v, axis=1, keepdims=True)
        o_ref[...] = v - jnp.log(jnp.sum(jnp.exp(v), axis=1, keepdims=True))

    return pl.pallas_call(
        body,
        grid=(N // _ROWS,),
        in_specs=[pl.BlockSpec((_ROWS, 1), lambda i: (i, 0)),
                  pl.BlockSpec((_ROWS, K), lambda i: (i, 0)),
                  pl.BlockSpec((_ROWS, K), lambda i: (i, 0)),
                  pl.BlockSpec((_ROWS, K), lambda i: (i, 0)),
                  pl.BlockSpec((1, K), lambda i: (0, 0))],
        out_specs=pl.BlockSpec((_ROWS, K), lambda i: (i, 0)),
        out_shape=jax.ShapeDtypeStruct((N, K), F32),
    )(dis, pa, pb, z2, b2)


def kernel(x, edge_index, edge_attr, W1, b1, W2, b2):
    N, Din = x.shape
    E = edge_index.shape[1]
    Dh = W1.shape[1]
    Dout = W2.shape[1]

    row = edge_index[0]
    col = edge_index[1]
    EP = -(-E // (CH * NW * 8)) * (CH * NW * 8)
    pad = EP - E
    if pad:
        prng = jnp.arange(pad, dtype=I32)
        row_sp = jnp.concatenate([row, prng % N]).reshape(EP // CH, CH)
        col2 = jnp.concatenate(
            [col, N + (prng % NT)]).reshape(EP // CH, CH)
        ew2 = jnp.concatenate(
            [edge_attr, jnp.zeros((pad,), F32)]).reshape(EP // CH, CH)
        row_dg = jnp.concatenate(
            [row, N + (prng % NT)]).reshape(EP // CH, CH)
    else:
        row_sp = row_dg = row.reshape(EP // CH, CH)
        col2 = col.reshape(EP // CH, CH)
        ew2 = edge_attr.reshape(EP // CH, CH)

    degp = _make_deg_kernel(N, EP)(row_dg).reshape(NC, N, L)
    y1 = _tc_matmul(x, W1)
    z1, dis = _tc_scale(degp[0], degp[1], y1)

    p1 = _make_spmm_kernel(N, EP, Dh)(
        z1, row_sp, col2, ew2).reshape(NC, N, Dh)
    z2 = _tc_layer2(dis, p1[0], p1[1], z1, b1.reshape(1, Dh), W2)

    p2 = _make_spmm_kernel(N, EP, Dout)(
        z2, row_sp, col2, ew2).reshape(NC, N, Dout)
    return _tc_out(dis, p2[0], p2[1], z2, b2.reshape(1, Dout))


# speedup vs baseline: 2.0791x; 1.0500x over previous
import functools

import jax
import jax.numpy as jnp
from jax import lax
from jax.experimental import pallas as pl
from jax.experimental.pallas import tpu as pltpu
from jax.experimental.pallas import tpu_sc as plsc

F32 = jnp.float32
I32 = jnp.int32

NC = 2
NS = 16
NW = NC * NS
L = 16

CH = 128


def _sc_mesh():
    return plsc.VectorSubcoreMesh(core_axis_name="c", subcore_axis_name="s")


NT = L


@functools.lru_cache(maxsize=None)
def _make_deg_kernel(N, EP):
    assert EP % (CH * NW) == 0 and N % NS == 0
    CPT = EP // CH // NW
    RPT = N // NS
    nfull = RPT // CH
    rem = RPT % CH

    @functools.partial(
        pl.kernel,
        mesh=_sc_mesh(),
        out_type=jax.ShapeDtypeStruct((NC * N, L), F32),
        compiler_params=pltpu.CompilerParams(use_tc_tiling_on_sc=False),
        scratch_types=[
            pltpu.VMEM((CPT, CH), I32),
            pltpu.VMEM((CH, L), F32),
            pltpu.VMEM_SHARED((N + NT, L), F32),
            pltpu.SemaphoreType.DMA,
        ],
    )
    def deg_kernel(row_hbm, out_hbm, ridx, ones_v, acc, sem):
        c = lax.axis_index("c")
        s = lax.axis_index("s")
        t = c * NS + s
        zeros = jnp.zeros((L,), F32)
        ones = jnp.ones((L,), F32)

        pltpu.sync_copy(row_hbm.at[pl.ds(t * CPT, CPT)], ridx)

        def zf(i, carry):
            ones_v[i, pl.ds(0, L)] = zeros
            return carry

        lax.fori_loop(0, CH, zf, None)
        for q in range(nfull):
            pltpu.sync_copy(ones_v, acc.at[pl.ds(s * RPT + q * CH, CH)])
        if rem:
            pltpu.sync_copy(ones_v.at[pl.ds(0, rem)],
                            acc.at[pl.ds(s * RPT + nfull * CH, rem)])
        if NT:
            @pl.when(s == 0)
            def _():
                pltpu.sync_copy(ones_v.at[pl.ds(0, NT)],
                                acc.at[pl.ds(N, NT)])

        def of(i, carry):
            ones_v[i, pl.ds(0, L)] = ones
            return carry

        lax.fori_loop(0, CH, of, None)
        plsc.subcore_barrier()

        def fire(k, carry):
            pltpu.async_copy(ones_v, acc.at[ridx.at[k]], sem, add=True)
            return carry

        lax.fori_loop(0, CPT, fire, None)

        def drain(k, carry):
            pltpu.make_async_copy(ones_v, acc.at[ridx.at[0]], sem).wait()
            return carry

        lax.fori_loop(0, CPT, drain, None)
        plsc.subcore_barrier()

        for q in range(nfull):
            pltpu.sync_copy(acc.at[pl.ds(s * RPT + q * CH, CH)], ones_v)
            pltpu.sync_copy(ones_v,
                            out_hbm.at[pl.ds(c * N + s * RPT + q * CH, CH)])
        if rem:
            pltpu.sync_copy(acc.at[pl.ds(s * RPT + nfull * CH, rem)],
                            ones_v.at[pl.ds(0, rem)])
            pltpu.sync_copy(
                ones_v.at[pl.ds(0, rem)],
                out_hbm.at[pl.ds(c * N + s * RPT + nfull * CH, rem)])

    return deg_kernel


@functools.lru_cache(maxsize=None)
def _make_spmm_kernel(N, EP, D):
    CHS = 64
    GRP = 4 if D >= 128 else 8
    NH = 2
    assert EP % (CHS * NW) == 0 and D % L == 0 and N % NS == 0
    CPT = EP // CHS // NW
    HC = CPT // NH
    assert CPT % NH == 0 and HC % GRP == 0
    RPT = N // NS
    nfull = RPT // CHS
    rem = RPT % CHS

    @functools.partial(
        pl.kernel,
        mesh=_sc_mesh(),
        out_type=jax.ShapeDtypeStruct((NC * N, D), F32),
        compiler_params=pltpu.CompilerParams(use_tc_tiling_on_sc=False),
        scratch_types=[
            pltpu.VMEM((HC, CHS), I32),
            pltpu.VMEM((HC, CHS), I32),
            pltpu.VMEM((HC, CHS), F32),
            pltpu.VMEM((GRP, CHS, D), F32),
            pltpu.VMEM_SHARED((N + NT, D), F32),
        ] + [pltpu.SemaphoreType.DMA] * GRP,
    )
    def spmm_kernel(z_hbm, row_hbm, col_hbm, ew_hbm, out_hbm,
                    ridx, cidx, ewv, rows, acc, *sems):
        c = lax.axis_index("c")
        s = lax.axis_index("s")
        t = c * NS + s
        zeros = jnp.zeros((L,), F32)

        def zr(i, carry):
            for j in range(D // L):
                rows[0, i, pl.ds(j * L, L)] = zeros
            return carry

        lax.fori_loop(0, CHS, zr, None)

        for q in range(nfull):
            pltpu.sync_copy(rows.at[0], acc.at[pl.ds(s * RPT + q * CHS, CHS)])
        if rem:
            pltpu.sync_copy(rows.at[0].at[pl.ds(0, rem)],
                            acc.at[pl.ds(s * RPT + nfull * CHS, rem)])
        plsc.subcore_barrier()

        def scale_rows(b, k):
            @plsc.parallel_loop(0, CHS // L, step=1)
            def scale(g):
                wv = ewv[k, pl.ds(g * L, L)]
                for lidx in range(L):
                    w = wv[lidx]
                    r = g * L + lidx
                    for j in range(D // L):
                        rows[b, r, pl.ds(j * L, L)] = \
                            rows[b, r, pl.ds(j * L, L)] * w

        def group_body(gq, carry):
            k0 = gq * GRP
            ds = [pltpu.async_copy(z_hbm.at[ridx.at[k0 + b]], rows.at[b],
                                   sems[b]) for b in range(GRP)]
            for b in range(GRP):
                ds[b].wait()
                scale_rows(b, k0 + b)
                pltpu.sync_copy(rows.at[b], acc.at[cidx.at[k0 + b]],
                                add=True)
            return carry

        for h in range(NH):
            start = t * CPT + h * HC
            pltpu.sync_copy(row_hbm.at[pl.ds(start, HC)], ridx)
            pltpu.sync_copy(col_hbm.at[pl.ds(start, HC)], cidx)
            pltpu.sync_copy(ew_hbm.at[pl.ds(start, HC)], ewv)
            lax.fori_loop(0, HC // GRP, group_body, None)
        plsc.subcore_barrier()

        for q in range(nfull):
            pltpu.sync_copy(acc.at[pl.ds(s * RPT + q * CHS, CHS)], rows.at[0])
            pltpu.sync_copy(rows.at[0],
                            out_hbm.at[pl.ds(c * N + s * RPT + q * CHS, CHS)])
        if rem:
            pltpu.sync_copy(acc.at[pl.ds(s * RPT + nfull * CHS, rem)],
                            rows.at[0].at[pl.ds(0, rem)])
            pltpu.sync_copy(
                rows.at[0].at[pl.ds(0, rem)],
                out_hbm.at[pl.ds(c * N + s * RPT + nfull * CHS, rem)])

    return spmm_kernel


_ROWS = 1000


def _tc_matmul(x, W):
    N, Din = x.shape
    K = W.shape[1]

    def body(x_ref, w_ref, o_ref):
        o_ref[...] = jnp.dot(x_ref[...], w_ref[...],
                             preferred_element_type=F32)

    return pl.pallas_call(
        body,
        grid=(N // _ROWS,),
        in_specs=[pl.BlockSpec((_ROWS, Din), lambda i: (i, 0)),
                  pl.BlockSpec((Din, K), lambda i: (0, 0))],
        out_specs=pl.BlockSpec((_ROWS, K), lambda i: (i, 0)),
        out_shape=jax.ShapeDtypeStruct((N, K), F32),
    )(x, W)


def _tc_scale(degp0, degp1, y1):
    N, D = y1.shape

    def body(d0_ref, d1_ref, y_ref, z_ref, dis_ref):
        d = d0_ref[...] + d1_ref[...] + 1.0
        dis = lax.rsqrt(d)[:, 0:1]
        dis_ref[...] = dis
        z_ref[...] = y_ref[...] * dis

    return pl.pallas_call(
        body,
        grid=(N // _ROWS,),
        in_specs=[pl.BlockSpec((_ROWS, L), lambda i: (i, 0)),
                  pl.BlockSpec((_ROWS, L), lambda i: (i, 0)),
                  pl.BlockSpec((_ROWS, D), lambda i: (i, 0))],
        out_specs=[pl.BlockSpec((_ROWS, D), lambda i: (i, 0)),
                   pl.BlockSpec((_ROWS, 1), lambda i: (i, 0))],
        out_shape=[jax.ShapeDtypeStruct((N, D), F32),
                   jax.ShapeDtypeStruct((N, 1), F32)],
    )(degp0, degp1, y1)


def _tc_layer2(dis, pa, pb, z1, b1, W2):
    N, D = z1.shape
    K = W2.shape[1]

    def body(dis_ref, pa_ref, pb_ref, z1_ref, b1_ref, w_ref, z2_ref):
        dis = dis_ref[...]
        h = (pa_ref[...] + pb_ref[...] + z1_ref[...]) * dis + b1_ref[...]
        h = jnp.maximum(h, 0.0)
        z2_ref[...] = jnp.dot(h, w_ref[...], preferred_element_type=F32) * dis

    return pl.pallas_call(
        body,
        grid=(N // _ROWS,),
        in_specs=[pl.BlockSpec((_ROWS, 1), lambda i: (i, 0)),
                  pl.BlockSpec((_ROWS, D), lambda i: (i, 0)),
                  pl.BlockSpec((_ROWS, D), lambda i: (i, 0)),
                  pl.BlockSpec((_ROWS, D), lambda i: (i, 0)),
                  pl.BlockSpec((1, D), lambda i: (0, 0)),
                  pl.BlockSpec((D, K), lambda i: (0, 0))],
        out_specs=pl.BlockSpec((_ROWS, K), lambda i: (i, 0)),
        out_shape=jax.ShapeDtypeStruct((N, K), F32),
    )(dis, pa, pb, z1, b1, W2)


def _tc_out(dis, pa, pb, z2, b2):
    N, K = z2.shape

    def body(dis_ref, pa_ref, pb_ref, z2_ref, b2_ref, o_ref):
        v = (pa_ref[...] + pb_ref[...] + z2_ref[...]) * dis_ref[...] \
            + b2_ref[...]
        v = v - jnp.max(v, axis=1, keepdims=True)
        o_ref[...] = v - jnp.log(jnp.sum(jnp.exp(v), axis=1, keepdims=True))

    return pl.pallas_call(
        body,
        grid=(N // _ROWS,),
        in_specs=[pl.BlockSpec((_ROWS, 1), lambda i: (i, 0)),
                  pl.BlockSpec((_ROWS, K), lambda i: (i, 0)),
                  pl.BlockSpec((_ROWS, K), lambda i: (i, 0)),
                  pl.BlockSpec((_ROWS, K), lambda i: (i, 0)),
                  pl.BlockSpec((1, K), lambda i: (0, 0))],
        out_specs=pl.BlockSpec((_ROWS, K), lambda i: (i, 0)),
        out_shape=jax.ShapeDtypeStruct((N, K), F32),
    )(dis, pa, pb, z2, b2)


def kernel(x, edge_index, edge_attr, W1, b1, W2, b2):
    N, Din = x.shape
    E = edge_index.shape[1]
    Dh = W1.shape[1]
    Dout = W2.shape[1]

    row = edge_index[0]
    col = edge_index[1]
    EP = -(-E // (CH * NW * 8)) * (CH * NW * 8)
    pad = EP - E
    CHS = 64
    if pad:
        prng = jnp.arange(pad, dtype=I32)
        row_sp = jnp.concatenate([row, prng % N]).reshape(EP // CHS, CHS)
        col2 = jnp.concatenate(
            [col, N + (prng % NT)]).reshape(EP // CHS, CHS)
        ew2 = jnp.concatenate(
            [edge_attr, jnp.zeros((pad,), F32)]).reshape(EP // CHS, CHS)
        row_dg = jnp.concatenate(
            [row, N + (prng % NT)]).reshape(EP // CH, CH)
    else:
        row_sp = row.reshape(EP // CHS, CHS)
        col2 = col.reshape(EP // CHS, CHS)
        ew2 = edge_attr.reshape(EP // CHS, CHS)
        row_dg = row.reshape(EP // CH, CH)

    degp = _make_deg_kernel(N, EP)(row_dg).reshape(NC, N, L)
    y1 = _tc_matmul(x, W1)
    z1, dis = _tc_scale(degp[0], degp[1], y1)

    p1 = _make_spmm_kernel(N, EP, Dh)(
        z1, row_sp, col2, ew2).reshape(NC, N, Dh)
    z2 = _tc_layer2(dis, p1[0], p1[1], z1, b1.reshape(1, Dh), W2)

    p2 = _make_spmm_kernel(N, EP, Dout)(
        z2, row_sp, col2, ew2).reshape(NC, N, Dout)
    return _tc_out(dis, p2[0], p2[1], z2, b2.reshape(1, Dout))


# speedup vs baseline: 2.1805x; 1.0488x over previous
import functools

import jax
import jax.numpy as jnp
from jax import lax
from jax.experimental import pallas as pl
from jax.experimental.pallas import tpu as pltpu
from jax.experimental.pallas import tpu_sc as plsc

F32 = jnp.float32
I32 = jnp.int32

NC = 2
NS = 16
NW = NC * NS
L = 16

CH = 128


def _sc_mesh():
    return plsc.VectorSubcoreMesh(core_axis_name="c", subcore_axis_name="s")


NT = L


@functools.lru_cache(maxsize=None)
def _make_deg_kernel(N, EP):
    assert EP % (CH * NW) == 0 and N % NS == 0
    CPT = EP // CH // NW
    RPT = N // NS
    nfull = RPT // CH
    rem = RPT % CH

    @functools.partial(
        pl.kernel,
        mesh=_sc_mesh(),
        out_type=jax.ShapeDtypeStruct((NC * N, L), F32),
        compiler_params=pltpu.CompilerParams(use_tc_tiling_on_sc=False),
        scratch_types=[
            pltpu.VMEM((CPT, CH), I32),
            pltpu.VMEM((CH, L), F32),
            pltpu.VMEM_SHARED((N + NT, L), F32),
            pltpu.SemaphoreType.DMA,
        ],
    )
    def deg_kernel(row_hbm, out_hbm, ridx, ones_v, acc, sem):
        c = lax.axis_index("c")
        s = lax.axis_index("s")
        t = c * NS + s
        zeros = jnp.zeros((L,), F32)
        ones = jnp.ones((L,), F32)

        pltpu.sync_copy(row_hbm.at[pl.ds(t * CPT, CPT)], ridx)

        def zf(i, carry):
            ones_v[i, pl.ds(0, L)] = zeros
            return carry

        lax.fori_loop(0, CH, zf, None)
        for q in range(nfull):
            pltpu.sync_copy(ones_v, acc.at[pl.ds(s * RPT + q * CH, CH)])
        if rem:
            pltpu.sync_copy(ones_v.at[pl.ds(0, rem)],
                            acc.at[pl.ds(s * RPT + nfull * CH, rem)])
        if NT:
            @pl.when(s == 0)
            def _():
                pltpu.sync_copy(ones_v.at[pl.ds(0, NT)],
                                acc.at[pl.ds(N, NT)])

        def of(i, carry):
            ones_v[i, pl.ds(0, L)] = ones
            return carry

        lax.fori_loop(0, CH, of, None)
        plsc.subcore_barrier()

        def fire(k, carry):
            pltpu.async_copy(ones_v, acc.at[ridx.at[k]], sem, add=True)
            return carry

        lax.fori_loop(0, CPT, fire, None)

        def drain(k, carry):
            pltpu.make_async_copy(ones_v, acc.at[ridx.at[0]], sem).wait()
            return carry

        lax.fori_loop(0, CPT, drain, None)
        plsc.subcore_barrier()

        for q in range(nfull):
            pltpu.sync_copy(acc.at[pl.ds(s * RPT + q * CH, CH)], ones_v)
            pltpu.sync_copy(ones_v,
                            out_hbm.at[pl.ds(c * N + s * RPT + q * CH, CH)])
        if rem:
            pltpu.sync_copy(acc.at[pl.ds(s * RPT + nfull * CH, rem)],
                            ones_v.at[pl.ds(0, rem)])
            pltpu.sync_copy(
                ones_v.at[pl.ds(0, rem)],
                out_hbm.at[pl.ds(c * N + s * RPT + nfull * CH, rem)])

    return deg_kernel


@functools.lru_cache(maxsize=None)
def _make_spmm_kernel(N, EP, D):
    tc_tiled = D % 128 == 0
    if tc_tiled:
        CHS, GRP, NH = 128, 2, 2
    else:
        CHS, GRP, NH = 64, 8, 2
    assert EP % (CHS * NW * NH * GRP) == 0 and D % L == 0
    CPT = EP // CHS // NW
    HC = CPT // NH
    if tc_tiled:
        starts = [624 * t for t in range(NS)] + [N]
    else:
        starts = [(N // NS) * t for t in range(NS)] + [N]
    wb = []
    for t in range(NS):
        span = starts[t + 1] - starts[t]
        sizes = [CHS] * (span // CHS) + ([span % CHS] if span % CHS else [])
        wb.append(sizes)
    assert all(starts[t] % 8 == 0 for t in range(NS)) or not tc_tiled

    @functools.partial(
        pl.kernel,
        mesh=_sc_mesh(),
        out_type=jax.ShapeDtypeStruct((NC * N, D), F32),
        compiler_params=(None if tc_tiled else
                         pltpu.CompilerParams(use_tc_tiling_on_sc=False)),
        scratch_types=[
            pltpu.VMEM((HC, CHS), I32),
            pltpu.VMEM((HC, CHS), I32),
            pltpu.VMEM((HC, CHS), F32),
            pltpu.VMEM((GRP, CHS, D), F32),
            pltpu.VMEM_SHARED((N + NT, D), F32),
        ] + [pltpu.SemaphoreType.DMA] * GRP,
    )
    def spmm_kernel(z_hbm, row_hbm, col_hbm, ew_hbm, out_hbm,
                    ridx, cidx, ewv, rows, acc, *sems):
        c = lax.axis_index("c")
        s = lax.axis_index("s")
        t = c * NS + s
        zeros = jnp.zeros((L,), F32)

        def zr(i, carry):
            for j in range(D // L):
                rows[0, i, pl.ds(j * L, L)] = zeros
            return carry

        lax.fori_loop(0, CHS, zr, None)

        for ts in range(NS):
            @pl.when(s == ts)
            def _(ts=ts):
                off = starts[ts]
                for sz in wb[ts]:
                    pltpu.sync_copy(rows.at[0].at[pl.ds(0, sz)],
                                    acc.at[pl.ds(off, sz)])
                    off += sz
        plsc.subcore_barrier()

        def scale_rows(b, k):
            @plsc.parallel_loop(0, CHS // L, step=1)
            def scale(g):
                wv = ewv[k, pl.ds(g * L, L)]
                for lidx in range(L):
                    w = wv[lidx]
                    r = g * L + lidx
                    for j in range(D // L):
                        rows[b, r, pl.ds(j * L, L)] = \
                            rows[b, r, pl.ds(j * L, L)] * w

        def group_body(gq, carry):
            k0 = gq * GRP
            ds = [pltpu.async_copy(z_hbm.at[ridx.at[k0 + b]], rows.at[b],
                                   sems[b]) for b in range(GRP)]
            for b in range(GRP):
                ds[b].wait()
                scale_rows(b, k0 + b)
                pltpu.sync_copy(rows.at[b], acc.at[cidx.at[k0 + b]],
                                add=True)
            return carry

        for h in range(NH):
            start = t * CPT + h * HC
            pltpu.sync_copy(row_hbm.at[pl.ds(start, HC)], ridx)
            pltpu.sync_copy(col_hbm.at[pl.ds(start, HC)], cidx)
            pltpu.sync_copy(ew_hbm.at[pl.ds(start, HC)], ewv)
            lax.fori_loop(0, HC // GRP, group_body, None)
        plsc.subcore_barrier()

        for ts in range(NS):
            @pl.when(s == ts)
            def _(ts=ts):
                off = starts[ts]
                for sz in wb[ts]:
                    pltpu.sync_copy(acc.at[pl.ds(off, sz)],
                                    rows.at[0].at[pl.ds(0, sz)])
                    pltpu.sync_copy(rows.at[0].at[pl.ds(0, sz)],
                                    out_hbm.at[pl.ds(c * N + off, sz)])
                    off += sz
        plsc.subcore_barrier()

    return spmm_kernel


_ROWS = 1000


def _tc_matmul(x, W):
    N, Din = x.shape
    K = W.shape[1]

    def body(x_ref, w_ref, o_ref):
        o_ref[...] = jnp.dot(x_ref[...], w_ref[...],
                             preferred_element_type=F32)

    return pl.pallas_call(
        body,
        grid=(N // _ROWS,),
        in_specs=[pl.BlockSpec((_ROWS, Din), lambda i: (i, 0)),
                  pl.BlockSpec((Din, K), lambda i: (0, 0))],
        out_specs=pl.BlockSpec((_ROWS, K), lambda i: (i, 0)),
        out_shape=jax.ShapeDtypeStruct((N, K), F32),
    )(x, W)


def _tc_scale(degp0, degp1, y1):
    N, D = y1.shape

    def body(d0_ref, d1_ref, y_ref, z_ref, dis_ref):
        d = d0_ref[...] + d1_ref[...] + 1.0
        dis = lax.rsqrt(d)[:, 0:1]
        dis_ref[...] = dis
        z_ref[...] = y_ref[...] * dis

    return pl.pallas_call(
        body,
        grid=(N // _ROWS,),
        in_specs=[pl.BlockSpec((_ROWS, L), lambda i: (i, 0)),
                  pl.BlockSpec((_ROWS, L), lambda i: (i, 0)),
                  pl.BlockSpec((_ROWS, D), lambda i: (i, 0))],
        out_specs=[pl.BlockSpec((_ROWS, D), lambda i: (i, 0)),
                   pl.BlockSpec((_ROWS, 1), lambda i: (i, 0))],
        out_shape=[jax.ShapeDtypeStruct((N, D), F32),
                   jax.ShapeDtypeStruct((N, 1), F32)],
    )(degp0, degp1, y1)


def _tc_layer2(dis, pa, pb, z1, b1, W2):
    N, D = z1.shape
    K = W2.shape[1]

    def body(dis_ref, pa_ref, pb_ref, z1_ref, b1_ref, w_ref, z2_ref):
        dis = dis_ref[...]
        h = (pa_ref[...] + pb_ref[...] + z1_ref[...]) * dis + b1_ref[...]
        h = jnp.maximum(h, 0.0)
        z2_ref[...] = jnp.dot(h, w_ref[...], preferred_element_type=F32) * dis

    return pl.pallas_call(
        body,
        grid=(N // _ROWS,),
        in_specs=[pl.BlockSpec((_ROWS, 1), lambda i: (i, 0)),
                  pl.BlockSpec((_ROWS, D), lambda i: (i, 0)),
                  pl.BlockSpec((_ROWS, D), lambda i: (i, 0)),
                  pl.BlockSpec((_ROWS, D), lambda i: (i, 0)),
                  pl.BlockSpec((1, D), lambda i: (0, 0)),
                  pl.BlockSpec((D, K), lambda i: (0, 0))],
        out_specs=pl.BlockSpec((_ROWS, K), lambda i: (i, 0)),
        out_shape=jax.ShapeDtypeStruct((N, K), F32),
    )(dis, pa, pb, z1, b1, W2)


def _tc_out(dis, pa, pb, z2, b2):
    N, K = z2.shape

    def body(dis_ref, pa_ref, pb_ref, z2_ref, b2_ref, o_ref):
        v = (pa_ref[...] + pb_ref[...] + z2_ref[...]) * dis_ref[...] \
            + b2_ref[...]
        v = v - jnp.max(v, axis=1, keepdims=True)
        o_ref[...] = v - jnp.log(jnp.sum(jnp.exp(v), axis=1, keepdims=True))

    return pl.pallas_call(
        body,
        grid=(N // _ROWS,),
        in_specs=[pl.BlockSpec((_ROWS, 1), lambda i: (i, 0)),
                  pl.BlockSpec((_ROWS, K), lambda i: (i, 0)),
                  pl.BlockSpec((_ROWS, K), lambda i: (i, 0)),
                  pl.BlockSpec((_ROWS, K), lambda i: (i, 0)),
                  pl.BlockSpec((1, K), lambda i: (0, 0))],
        out_specs=pl.BlockSpec((_ROWS, K), lambda i: (i, 0)),
        out_shape=jax.ShapeDtypeStruct((N, K), F32),
    )(dis, pa, pb, z2, b2)


def kernel(x, edge_index, edge_attr, W1, b1, W2, b2):
    N, Din = x.shape
    E = edge_index.shape[1]
    Dh = W1.shape[1]
    Dout = W2.shape[1]

    row = edge_index[0]
    col = edge_index[1]
    EP = -(-E // (CH * NW * 8)) * (CH * NW * 8)
    pad = EP - E
    if pad:
        prng = jnp.arange(pad, dtype=I32)
        row_sp = jnp.concatenate([row, prng % N])
        col_sp = jnp.concatenate([col, N + (prng % NT)])
        ew_sp = jnp.concatenate([edge_attr, jnp.zeros((pad,), F32)])
        row_dg = jnp.concatenate([row, N + (prng % NT)])
    else:
        row_sp = row_dg = row
        col_sp = col
        ew_sp = edge_attr

    def chunks(a, w):
        return a.reshape(EP // w, w)

    degp = _make_deg_kernel(N, EP)(chunks(row_dg, CH)).reshape(NC, N, L)
    y1 = _tc_matmul(x, W1)
    z1, dis = _tc_scale(degp[0], degp[1], y1)

    W1CH, W2CH = (128 if Dh % 128 == 0 else 64), (128 if Dout % 128 == 0
                                                  else 64)
    p1 = _make_spmm_kernel(N, EP, Dh)(
        z1, chunks(row_sp, W1CH), chunks(col_sp, W1CH),
        chunks(ew_sp, W1CH)).reshape(NC, N, Dh)
    z2 = _tc_layer2(dis, p1[0], p1[1], z1, b1.reshape(1, Dh), W2)

    p2 = _make_spmm_kernel(N, EP, Dout)(
        z2, chunks(row_sp, W2CH), chunks(col_sp, W2CH),
        chunks(ew_sp, W2CH)).reshape(NC, N, Dout)
    return _tc_out(dis, p2[0], p2[1], z2, b2.reshape(1, Dout))


# speedup vs baseline: 2.6290x; 1.2057x over previous
import functools

import jax
import jax.numpy as jnp
from jax import lax
from jax.experimental import pallas as pl
from jax.experimental.pallas import tpu as pltpu
from jax.experimental.pallas import tpu_sc as plsc

F32 = jnp.float32
I32 = jnp.int32

NC = 2
NS = 16
NW = NC * NS
L = 16

CH = 128


def _sc_mesh():
    return plsc.VectorSubcoreMesh(core_axis_name="c", subcore_axis_name="s")


NT = L


@functools.lru_cache(maxsize=None)
def _make_deg_kernel(N, EP):
    assert EP % (CH * NW) == 0 and N % NS == 0
    CPT = EP // CH // NW
    RPT = N // NS
    nfull = RPT // CH
    rem = RPT % CH

    @functools.partial(
        pl.kernel,
        mesh=_sc_mesh(),
        out_type=jax.ShapeDtypeStruct((NC * N, L), F32),
        compiler_params=pltpu.CompilerParams(use_tc_tiling_on_sc=False),
        scratch_types=[
            pltpu.VMEM((CPT, CH), I32),
            pltpu.VMEM((CH, L), F32),
            pltpu.VMEM_SHARED((N + NT, L), F32),
            pltpu.SemaphoreType.DMA,
        ],
    )
    def deg_kernel(row_hbm, out_hbm, ridx, ones_v, acc, sem):
        c = lax.axis_index("c")
        s = lax.axis_index("s")
        t = c * NS + s
        zeros = jnp.zeros((L,), F32)
        ones = jnp.ones((L,), F32)

        pltpu.sync_copy(row_hbm.at[pl.ds(t * CPT, CPT)], ridx)

        def zf(i, carry):
            ones_v[i, pl.ds(0, L)] = zeros
            return carry

        lax.fori_loop(0, CH, zf, None)
        for q in range(nfull):
            pltpu.sync_copy(ones_v, acc.at[pl.ds(s * RPT + q * CH, CH)])
        if rem:
            pltpu.sync_copy(ones_v.at[pl.ds(0, rem)],
                            acc.at[pl.ds(s * RPT + nfull * CH, rem)])
        if NT:
            @pl.when(s == 0)
            def _():
                pltpu.sync_copy(ones_v.at[pl.ds(0, NT)],
                                acc.at[pl.ds(N, NT)])

        def of(i, carry):
            ones_v[i, pl.ds(0, L)] = ones
            return carry

        lax.fori_loop(0, CH, of, None)
        plsc.subcore_barrier()

        def fire(k, carry):
            pltpu.async_copy(ones_v, acc.at[ridx.at[k]], sem, add=True)
            return carry

        lax.fori_loop(0, CPT, fire, None)

        def drain(k, carry):
            pltpu.make_async_copy(ones_v, acc.at[ridx.at[0]], sem).wait()
            return carry

        lax.fori_loop(0, CPT, drain, None)
        plsc.subcore_barrier()

        for q in range(nfull):
            pltpu.sync_copy(acc.at[pl.ds(s * RPT + q * CH, CH)], ones_v)
            pltpu.sync_copy(ones_v,
                            out_hbm.at[pl.ds(c * N + s * RPT + q * CH, CH)])
        if rem:
            pltpu.sync_copy(acc.at[pl.ds(s * RPT + nfull * CH, rem)],
                            ones_v.at[pl.ds(0, rem)])
            pltpu.sync_copy(
                ones_v.at[pl.ds(0, rem)],
                out_hbm.at[pl.ds(c * N + s * RPT + nfull * CH, rem)])

    return deg_kernel


@functools.lru_cache(maxsize=None)
def _make_spmm_kernel(N, EP, D):
    tc_tiled = D % 128 == 0
    if tc_tiled:
        CHS, GRP, NH = 128, 2, 2
    else:
        CHS, GRP, NH = 64, 8, 2
    assert EP % (CHS * NW * NH * GRP) == 0 and D % L == 0
    CPT = EP // CHS // NW
    HC = CPT // NH
    if tc_tiled:
        starts = [624 * t for t in range(NS)] + [N]
    else:
        starts = [(N // NS) * t for t in range(NS)] + [N]
    wb = []
    for t in range(NS):
        span = starts[t + 1] - starts[t]
        sizes = [CHS] * (span // CHS) + ([span % CHS] if span % CHS else [])
        wb.append(sizes)
    assert all(starts[t] % 8 == 0 for t in range(NS)) or not tc_tiled

    @functools.partial(
        pl.kernel,
        mesh=_sc_mesh(),
        out_type=jax.ShapeDtypeStruct((NC * N, D), F32),
        compiler_params=(None if tc_tiled else
                         pltpu.CompilerParams(use_tc_tiling_on_sc=False)),
        scratch_types=[
            pltpu.VMEM((HC, CHS), I32),
            pltpu.VMEM((HC, CHS), I32),
            pltpu.VMEM((HC, CHS), F32),
            pltpu.VMEM((GRP, CHS, D), F32),
            pltpu.VMEM_SHARED((N + NT, D), F32),
        ] + [pltpu.SemaphoreType.DMA] * (2 * GRP),
    )
    def spmm_kernel(z_hbm, row_hbm, col_hbm, ew_hbm, out_hbm,
                    ridx, cidx, ewv, rows, acc, *allsems):
        sems, ssems = allsems[:GRP], allsems[GRP:]
        c = lax.axis_index("c")
        s = lax.axis_index("s")
        t = c * NS + s
        zeros = jnp.zeros((L,), F32)

        def zr(i, carry):
            for j in range(D // L):
                rows[0, i, pl.ds(j * L, L)] = zeros
            return carry

        lax.fori_loop(0, CHS, zr, None)

        for ts in range(NS):
            @pl.when(s == ts)
            def _(ts=ts):
                off = starts[ts]
                for sz in wb[ts]:
                    pltpu.sync_copy(rows.at[0].at[pl.ds(0, sz)],
                                    acc.at[pl.ds(off, sz)])
                    off += sz
        plsc.subcore_barrier()

        def scale_rows(b, k):
            @plsc.parallel_loop(0, CHS // L, step=1)
            def scale(g):
                wv = ewv[k, pl.ds(g * L, L)]
                for lidx in range(L):
                    w = wv[lidx]
                    r = g * L + lidx
                    for j in range(D // L):
                        rows[b, r, pl.ds(j * L, L)] = \
                            rows[b, r, pl.ds(j * L, L)] * w

        def drain_scatters():
            for b in range(GRP):
                pltpu.make_async_copy(rows.at[b], acc.at[cidx.at[0]],
                                      ssems[b]).wait()

        def group_body(gq, carry):
            k0 = gq * GRP

            @pl.when(gq > 0)
            def _():
                drain_scatters()

            ds = [pltpu.async_copy(z_hbm.at[ridx.at[k0 + b]], rows.at[b],
                                   sems[b]) for b in range(GRP)]
            for b in range(GRP):
                ds[b].wait()
                scale_rows(b, k0 + b)
                pltpu.async_copy(rows.at[b], acc.at[cidx.at[k0 + b]],
                                 ssems[b], add=True)
            return carry

        for h in range(NH):
            start = t * CPT + h * HC
            pltpu.sync_copy(row_hbm.at[pl.ds(start, HC)], ridx)
            pltpu.sync_copy(col_hbm.at[pl.ds(start, HC)], cidx)
            pltpu.sync_copy(ew_hbm.at[pl.ds(start, HC)], ewv)
            lax.fori_loop(0, HC // GRP, group_body, None)
            drain_scatters()
        plsc.subcore_barrier()

        for ts in range(NS):
            @pl.when(s == ts)
            def _(ts=ts):
                off = starts[ts]
                for sz in wb[ts]:
                    pltpu.sync_copy(acc.at[pl.ds(off, sz)],
                                    rows.at[0].at[pl.ds(0, sz)])
                    pltpu.sync_copy(rows.at[0].at[pl.ds(0, sz)],
                                    out_hbm.at[pl.ds(c * N + off, sz)])
                    off += sz
        plsc.subcore_barrier()

    return spmm_kernel


_ROWS = 2000


def _tc_matmul(x, W):
    N, Din = x.shape
    K = W.shape[1]

    def body(x_ref, w_ref, o_ref):
        o_ref[...] = jnp.dot(x_ref[...], w_ref[...],
                             preferred_element_type=F32)

    return pl.pallas_call(
        body,
        grid=(N // _ROWS,),
        in_specs=[pl.BlockSpec((_ROWS, Din), lambda i: (i, 0)),
                  pl.BlockSpec((Din, K), lambda i: (0, 0))],
        out_specs=pl.BlockSpec((_ROWS, K), lambda i: (i, 0)),
        out_shape=jax.ShapeDtypeStruct((N, K), F32),
    )(x, W)


def _pair_specs(R, K, nblk):
    return [pl.BlockSpec((R, K), lambda i: (i, 0)),
            pl.BlockSpec((R, K), lambda i, _n=nblk: (i + _n, 0))]


def _tc_scale(degp, y1):
    N, D = y1.shape
    nblk = N // _ROWS

    def body(d0_ref, d1_ref, y_ref, z_ref, dis_ref):
        d = d0_ref[...] + d1_ref[...] + 1.0
        dis = lax.rsqrt(d)[:, 0:1]
        dis_ref[...] = dis
        z_ref[...] = y_ref[...] * dis

    return pl.pallas_call(
        body,
        grid=(nblk,),
        in_specs=_pair_specs(_ROWS, L, nblk) +
        [pl.BlockSpec((_ROWS, D), lambda i: (i, 0))],
        out_specs=[pl.BlockSpec((_ROWS, D), lambda i: (i, 0)),
                   pl.BlockSpec((_ROWS, 1), lambda i: (i, 0))],
        out_shape=[jax.ShapeDtypeStruct((N, D), F32),
                   jax.ShapeDtypeStruct((N, 1), F32)],
    )(degp, degp, y1)


def _tc_layer2(dis, p1, z1, b1, W2):
    N, D = z1.shape
    K = W2.shape[1]
    nblk = N // _ROWS

    def body(dis_ref, pa_ref, pb_ref, z1_ref, b1_ref, w_ref, z2_ref):
        dis = dis_ref[...]
        h = (pa_ref[...] + pb_ref[...] + z1_ref[...]) * dis + b1_ref[...]
        h = jnp.maximum(h, 0.0)
        z2_ref[...] = jnp.dot(h, w_ref[...], preferred_element_type=F32) * dis

    return pl.pallas_call(
        body,
        grid=(nblk,),
        in_specs=[pl.BlockSpec((_ROWS, 1), lambda i: (i, 0))] +
        _pair_specs(_ROWS, D, nblk) +
        [pl.BlockSpec((_ROWS, D), lambda i: (i, 0)),
         pl.BlockSpec((1, D), lambda i: (0, 0)),
         pl.BlockSpec((D, K), lambda i: (0, 0))],
        out_specs=pl.BlockSpec((_ROWS, K), lambda i: (i, 0)),
        out_shape=jax.ShapeDtypeStruct((N, K), F32),
    )(dis, p1, p1, z1, b1, W2)


def _tc_out(dis, p2, z2, b2):
    N, K = z2.shape
    nblk = N // _ROWS

    def body(dis_ref, pa_ref, pb_ref, z2_ref, b2_ref, o_ref):
        v = (pa_ref[...] + pb_ref[...] + z2_ref[...]) * dis_ref[...] \
            + b2_ref[...]
        v = v - jnp.max(v, axis=1, keepdims=True)
        o_ref[...] = v - jnp.log(jnp.sum(jnp.exp(v), axis=1, keepdims=True))

    return pl.pallas_call(
        body,
        grid=(nblk,),
        in_specs=[pl.BlockSpec((_ROWS, 1), lambda i: (i, 0))] +
        _pair_specs(_ROWS, K, nblk) +
        [pl.BlockSpec((_ROWS, K), lambda i: (i, 0)),
         pl.BlockSpec((1, K), lambda i: (0, 0))],
        out_specs=pl.BlockSpec((_ROWS, K), lambda i: (i, 0)),
        out_shape=jax.ShapeDtypeStruct((N, K), F32),
    )(dis, p2, p2, z2, b2)


def kernel(x, edge_index, edge_attr, W1, b1, W2, b2):
    N, Din = x.shape
    E = edge_index.shape[1]
    Dh = W1.shape[1]
    Dout = W2.shape[1]

    row = edge_index[0]
    col = edge_index[1]
    EP = -(-E // (CH * NW * 8)) * (CH * NW * 8)
    pad = EP - E
    if pad:
        prng = jnp.arange(pad, dtype=I32)
        row_sp = jnp.concatenate([row, prng % N])
        col_sp = jnp.concatenate([col, N + (prng % NT)])
        ew_sp = jnp.concatenate([edge_attr, jnp.zeros((pad,), F32)])
        row_dg = jnp.concatenate([row, N + (prng % NT)])
    else:
        row_sp = row_dg = row
        col_sp = col
        ew_sp = edge_attr

    def chunks(a, w):
        return a.reshape(EP // w, w)

    degp = _make_deg_kernel(N, EP)(chunks(row_dg, CH))
    y1 = _tc_matmul(x, W1)
    z1, dis = _tc_scale(degp, y1)

    W1CH, W2CH = (128 if Dh % 128 == 0 else 64), (128 if Dout % 128 == 0
                                                  else 64)
    p1 = _make_spmm_kernel(N, EP, Dh)(
        z1, chunks(row_sp, W1CH), chunks(col_sp, W1CH), chunks(ew_sp, W1CH))
    z2 = _tc_layer2(dis, p1, z1, b1.reshape(1, Dh), W2)

    p2 = _make_spmm_kernel(N, EP, Dout)(
        z2, chunks(row_sp, W2CH), chunks(col_sp, W2CH), chunks(ew_sp, W2CH))
    return _tc_out(dis, p2, z2, b2.reshape(1, Dout))


# speedup vs baseline: 3.0109x; 1.1453x over previous
import functools

import jax
import jax.numpy as jnp
from jax import lax
from jax.experimental import pallas as pl
from jax.experimental.pallas import tpu as pltpu
from jax.experimental.pallas import tpu_sc as plsc

F32 = jnp.float32
I32 = jnp.int32

NC = 2
NS = 16
NW = NC * NS
L = 16

CH = 128


def _sc_mesh():
    return plsc.VectorSubcoreMesh(core_axis_name="c", subcore_axis_name="s")


NT = L


@functools.lru_cache(maxsize=None)
def _make_deg_kernel(N, EP):
    assert EP % (CH * NW) == 0 and N % NS == 0
    CPT = EP // CH // NW
    RPT = N // NS
    nfull = RPT // CH
    rem = RPT % CH

    @functools.partial(
        pl.kernel,
        mesh=_sc_mesh(),
        out_type=jax.ShapeDtypeStruct((NC * N, L), F32),
        compiler_params=pltpu.CompilerParams(use_tc_tiling_on_sc=False),
        scratch_types=[
            pltpu.VMEM((CPT, CH), I32),
            pltpu.VMEM((CH, L), F32),
            pltpu.VMEM_SHARED((N + NT, L), F32),
            pltpu.SemaphoreType.DMA,
        ],
    )
    def deg_kernel(row_hbm, out_hbm, ridx, ones_v, acc, sem):
        c = lax.axis_index("c")
        s = lax.axis_index("s")
        t = c * NS + s
        zeros = jnp.zeros((L,), F32)
        ones = jnp.ones((L,), F32)

        pltpu.sync_copy(row_hbm.at[pl.ds(t * CPT, CPT)], ridx)

        def zf(i, carry):
            ones_v[i, pl.ds(0, L)] = zeros
            return carry

        lax.fori_loop(0, CH, zf, None)
        for q in range(nfull):
            pltpu.sync_copy(ones_v, acc.at[pl.ds(s * RPT + q * CH, CH)])
        if rem:
            pltpu.sync_copy(ones_v.at[pl.ds(0, rem)],
                            acc.at[pl.ds(s * RPT + nfull * CH, rem)])
        if NT:
            @pl.when(s == 0)
            def _():
                pltpu.sync_copy(ones_v.at[pl.ds(0, NT)],
                                acc.at[pl.ds(N, NT)])

        def of(i, carry):
            ones_v[i, pl.ds(0, L)] = ones
            return carry

        lax.fori_loop(0, CH, of, None)
        plsc.subcore_barrier()

        def fire(k, carry):
            pltpu.async_copy(ones_v, acc.at[ridx.at[k]], sem, add=True)
            return carry

        lax.fori_loop(0, CPT, fire, None)

        def drain(k, carry):
            pltpu.make_async_copy(ones_v, acc.at[ridx.at[0]], sem).wait()
            return carry

        lax.fori_loop(0, CPT, drain, None)
        plsc.subcore_barrier()

        for q in range(nfull):
            pltpu.sync_copy(acc.at[pl.ds(s * RPT + q * CH, CH)], ones_v)
            pltpu.sync_copy(ones_v,
                            out_hbm.at[pl.ds(c * N + s * RPT + q * CH, CH)])
        if rem:
            pltpu.sync_copy(acc.at[pl.ds(s * RPT + nfull * CH, rem)],
                            ones_v.at[pl.ds(0, rem)])
            pltpu.sync_copy(
                ones_v.at[pl.ds(0, rem)],
                out_hbm.at[pl.ds(c * N + s * RPT + nfull * CH, rem)])

    return deg_kernel


@functools.lru_cache(maxsize=None)
def _make_spmm_kernel(N, EP, D):
    tc_tiled = D % 128 == 0
    if tc_tiled:
        CHS, GRP, NH = 128, 2, 2
    else:
        CHS, GRP, NH = 128, 5, 2
    assert EP % (CHS * NW * NH * GRP) == 0 and D % L == 0
    CPT = EP // CHS // NW
    HC = CPT // NH
    if tc_tiled:
        starts = [624 * t for t in range(NS)] + [N]
    else:
        starts = [(N // NS) * t for t in range(NS)] + [N]
    wb = []
    for t in range(NS):
        span = starts[t + 1] - starts[t]
        sizes = [CHS] * (span // CHS) + ([span % CHS] if span % CHS else [])
        wb.append(sizes)
    assert all(starts[t] % 8 == 0 for t in range(NS)) or not tc_tiled

    @functools.partial(
        pl.kernel,
        mesh=_sc_mesh(),
        out_type=jax.ShapeDtypeStruct((NC * N, D), F32),
        compiler_params=(None if tc_tiled else
                         pltpu.CompilerParams(use_tc_tiling_on_sc=False)),
        scratch_types=[
            pltpu.VMEM((HC, CHS), I32),
            pltpu.VMEM((HC, CHS), I32),
            pltpu.VMEM((HC, CHS), F32),
            pltpu.VMEM((GRP, CHS, D), F32),
            pltpu.VMEM_SHARED((N + NT, D), F32),
        ] + [pltpu.SemaphoreType.DMA] * (2 * GRP),
    )
    def spmm_kernel(z_hbm, row_hbm, col_hbm, ew_hbm, out_hbm,
                    ridx, cidx, ewv, rows, acc, *allsems):
        sems, ssems = allsems[:GRP], allsems[GRP:]
        c = lax.axis_index("c")
        s = lax.axis_index("s")
        t = c * NS + s
        zeros = jnp.zeros((L,), F32)

        def zr(i, carry):
            for j in range(D // L):
                rows[0, i, pl.ds(j * L, L)] = zeros
            return carry

        lax.fori_loop(0, CHS, zr, None)

        for ts in range(NS):
            @pl.when(s == ts)
            def _(ts=ts):
                off = starts[ts]
                for sz in wb[ts]:
                    pltpu.sync_copy(rows.at[0].at[pl.ds(0, sz)],
                                    acc.at[pl.ds(off, sz)])
                    off += sz
        plsc.subcore_barrier()

        def scale_rows(b, k):
            @plsc.parallel_loop(0, CHS // L, step=1)
            def scale(g):
                wv = ewv[k, pl.ds(g * L, L)]
                for lidx in range(L):
                    w = wv[lidx]
                    r = g * L + lidx
                    for j in range(D // L):
                        rows[b, r, pl.ds(j * L, L)] = \
                            rows[b, r, pl.ds(j * L, L)] * w

        def drain_scatter(b):
            pltpu.make_async_copy(rows.at[b], acc.at[cidx.at[0]],
                                  ssems[b]).wait()

        def drain_scatters():
            for b in range(GRP):
                drain_scatter(b)

        def group_body(gq, carry):
            k0 = gq * GRP

            ds = []
            for b in range(GRP):
                @pl.when(gq > 0)
                def _(b=b):
                    drain_scatter(b)
                ds.append(pltpu.async_copy(z_hbm.at[ridx.at[k0 + b]],
                                           rows.at[b], sems[b]))
            for b in range(GRP):
                ds[b].wait()
                scale_rows(b, k0 + b)
                pltpu.async_copy(rows.at[b], acc.at[cidx.at[k0 + b]],
                                 ssems[b], add=True)
            return carry

        for h in range(NH):
            start = t * CPT + h * HC
            pltpu.sync_copy(row_hbm.at[pl.ds(start, HC)], ridx)
            pltpu.sync_copy(col_hbm.at[pl.ds(start, HC)], cidx)
            pltpu.sync_copy(ew_hbm.at[pl.ds(start, HC)], ewv)
            lax.fori_loop(0, HC // GRP, group_body, None)
            drain_scatters()
        plsc.subcore_barrier()

        for ts in range(NS):
            @pl.when(s == ts)
            def _(ts=ts):
                off = starts[ts]
                for sz in wb[ts]:
                    pltpu.sync_copy(acc.at[pl.ds(off, sz)],
                                    rows.at[0].at[pl.ds(0, sz)])
                    pltpu.sync_copy(rows.at[0].at[pl.ds(0, sz)],
                                    out_hbm.at[pl.ds(c * N + off, sz)])
                    off += sz
        plsc.subcore_barrier()

    return spmm_kernel


_ROWS = 2000


def _tc_matmul(x, W):
    N, Din = x.shape
    K = W.shape[1]

    def body(x_ref, w_ref, o_ref):
        o_ref[...] = jnp.dot(x_ref[...], w_ref[...],
                             preferred_element_type=F32)

    return pl.pallas_call(
        body,
        grid=(N // _ROWS,),
        in_specs=[pl.BlockSpec((_ROWS, Din), lambda i: (i, 0)),
                  pl.BlockSpec((Din, K), lambda i: (0, 0))],
        out_specs=pl.BlockSpec((_ROWS, K), lambda i: (i, 0)),
        out_shape=jax.ShapeDtypeStruct((N, K), F32),
    )(x, W)


def _pair_specs(R, K, nblk):
    return [pl.BlockSpec((R, K), lambda i: (i, 0)),
            pl.BlockSpec((R, K), lambda i, _n=nblk: (i + _n, 0))]


def _tc_scale(degp, y1):
    N, D = y1.shape
    nblk = N // _ROWS

    def body(d0_ref, d1_ref, y_ref, z_ref, dis_ref):
        d = d0_ref[...] + d1_ref[...] + 1.0
        dis = lax.rsqrt(d)[:, 0:1]
        dis_ref[...] = dis
        z_ref[...] = y_ref[...] * dis

    return pl.pallas_call(
        body,
        grid=(nblk,),
        in_specs=_pair_specs(_ROWS, L, nblk) +
        [pl.BlockSpec((_ROWS, D), lambda i: (i, 0))],
        out_specs=[pl.BlockSpec((_ROWS, D), lambda i: (i, 0)),
                   pl.BlockSpec((_ROWS, 1), lambda i: (i, 0))],
        out_shape=[jax.ShapeDtypeStruct((N, D), F32),
                   jax.ShapeDtypeStruct((N, 1), F32)],
    )(degp, degp, y1)


def _tc_layer2(dis, p1, z1, b1, W2):
    N, D = z1.shape
    K = W2.shape[1]
    nblk = N // _ROWS

    def body(dis_ref, pa_ref, pb_ref, z1_ref, b1_ref, w_ref, z2_ref):
        dis = dis_ref[...]
        h = (pa_ref[...] + pb_ref[...] + z1_ref[...]) * dis + b1_ref[...]
        h = jnp.maximum(h, 0.0)
        z2_ref[...] = jnp.dot(h, w_ref[...], preferred_element_type=F32) * dis

    return pl.pallas_call(
        body,
        grid=(nblk,),
        in_specs=[pl.BlockSpec((_ROWS, 1), lambda i: (i, 0))] +
        _pair_specs(_ROWS, D, nblk) +
        [pl.BlockSpec((_ROWS, D), lambda i: (i, 0)),
         pl.BlockSpec((1, D), lambda i: (0, 0)),
         pl.BlockSpec((D, K), lambda i: (0, 0))],
        out_specs=pl.BlockSpec((_ROWS, K), lambda i: (i, 0)),
        out_shape=jax.ShapeDtypeStruct((N, K), F32),
    )(dis, p1, p1, z1, b1, W2)


def _tc_out(dis, p2, z2, b2):
    N, K = z2.shape
    nblk = N // _ROWS

    def body(dis_ref, pa_ref, pb_ref, z2_ref, b2_ref, o_ref):
        v = (pa_ref[...] + pb_ref[...] + z2_ref[...]) * dis_ref[...] \
            + b2_ref[...]
        v = v - jnp.max(v, axis=1, keepdims=True)
        o_ref[...] = v - jnp.log(jnp.sum(jnp.exp(v), axis=1, keepdims=True))

    return pl.pallas_call(
        body,
        grid=(nblk,),
        in_specs=[pl.BlockSpec((_ROWS, 1), lambda i: (i, 0))] +
        _pair_specs(_ROWS, K, nblk) +
        [pl.BlockSpec((_ROWS, K), lambda i: (i, 0)),
         pl.BlockSpec((1, K), lambda i: (0, 0))],
        out_specs=pl.BlockSpec((_ROWS, K), lambda i: (i, 0)),
        out_shape=jax.ShapeDtypeStruct((N, K), F32),
    )(dis, p2, p2, z2, b2)


def kernel(x, edge_index, edge_attr, W1, b1, W2, b2):
    N, Din = x.shape
    E = edge_index.shape[1]
    Dh = W1.shape[1]
    Dout = W2.shape[1]

    row = edge_index[0]
    col = edge_index[1]
    EP = -(-E // (CH * NW * 8)) * (CH * NW * 8)
    pad = EP - E
    if pad:
        prng = jnp.arange(pad, dtype=I32)
        row_sp = jnp.concatenate([row, prng % N])
        col_sp = jnp.concatenate([col, N + (prng % NT)])
        ew_sp = jnp.concatenate([edge_attr, jnp.zeros((pad,), F32)])
        row_dg = jnp.concatenate([row, N + (prng % NT)])
    else:
        row_sp = row_dg = row
        col_sp = col
        ew_sp = edge_attr

    def chunks(a, w):
        return a.reshape(EP // w, w)

    degp = _make_deg_kernel(N, EP)(chunks(row_dg, CH))
    y1 = _tc_matmul(x, W1)
    z1, dis = _tc_scale(degp, y1)

    W1CH, W2CH = 128, 128
    p1 = _make_spmm_kernel(N, EP, Dh)(
        z1, chunks(row_sp, W1CH), chunks(col_sp, W1CH), chunks(ew_sp, W1CH))
    z2 = _tc_layer2(dis, p1, z1, b1.reshape(1, Dh), W2)

    p2 = _make_spmm_kernel(N, EP, Dout)(
        z2, chunks(row_sp, W2CH), chunks(col_sp, W2CH), chunks(ew_sp, W2CH))
    return _tc_out(dis, p2, z2, b2.reshape(1, Dout))
